# Initial kernel scaffold; baseline (speedup 1.0000x reference)
import jax
import jax.numpy as jnp
from jax.experimental import pallas as pl


def kernel(outputs1, outputs2, mask1, mask2, W_s, b_s, gW1, gb1, gW2, gb2):
    raise NotImplementedError("write your pallas kernel here")


# speedup vs baseline: 12.6299x; 12.6299x over previous
import functools

import jax
import jax.numpy as jnp
from jax import lax
from jax.experimental import pallas as pl
from jax.experimental.pallas import tpu as pltpu

NEG_INF = float("-inf")


def _topk_sim_kernel(h1_ref, h2_ref, w_ref, b_ref, neg1_ref, neg2_ref,
                     out_ref, s_scr, p_scr, *, k, len1, len2):
    h1 = h1_ref[0]
    h2 = h2_ref[0]
    w = w_ref[0]
    b = b_ref[0, 0, 0]
    a = jnp.dot(h1, w, preferred_element_type=jnp.float32)
    s = lax.dot_general(a, h2, (((1,), (1,)), ((), ())),
                        preferred_element_type=jnp.float32)
    s = jnp.maximum(s + b, 0.0) + neg1_ref[0] + neg2_ref[0, 0][None, :]
    s_scr[...] = s
    p_scr[...] = jnp.max(s.reshape(len1 // 128, 128, len2), axis=2)

    rows16 = len1 // 128
    iota_ph = lax.broadcasted_iota(jnp.int32, (rows16, 128), 0)
    iota_pl = lax.broadcasted_iota(jnp.int32, (rows16, 128), 1)
    iota_row = iota_ph * 128 + iota_pl
    iota_col = lax.broadcasted_iota(jnp.int32, (1, len2), 1)
    iota_lane = lax.broadcasted_iota(jnp.int32, (1, k), 1)
    big = jnp.int32(len1 * len2)

    def body(i, _):
        p = p_scr[...]
        m = jnp.max(p)
        r = jnp.min(jnp.where(p == m, iota_row, big))
        row = s_scr[pl.ds(r, 1), :]
        j = jnp.min(jnp.where(row == m, iota_col, big))
        new_row = jnp.where(iota_col == j, NEG_INF, row)
        s_scr[pl.ds(r, 1), :] = new_row
        new_max = jnp.max(new_row)
        rh = r // 128
        rl = r % 128
        prow = p_scr[pl.ds(rh, 1), :]
        p_scr[pl.ds(rh, 1), :] = jnp.where(
            lax.broadcasted_iota(jnp.int32, (1, 128), 1) == rl, new_max, prow)
        out_ref[0, 0, 0:1, :] = jnp.where(iota_lane == i, m,
                                          out_ref[0, 0, 0:1, :])
        return 0

    lax.fori_loop(0, k, body, 0)

    q = out_ref[0, 0, 0:1, :]
    isneg = q == NEG_INF
    nf = k - jnp.sum(isneg.astype(jnp.int32))
    idx = lax.rem(lax.rem(nf - 1, k) + k, k)
    last_val = jnp.sum(jnp.where(iota_lane == idx, q, 0.0))
    out_ref[0, 0, 0:1, :] = jnp.where(isneg, last_val, q)


def _mlp_kernel(q_ref, w1_ref, b1_ref, w2_ref, b2_ref, out_ref):
    h = jnp.dot(q_ref[...], w1_ref[...], preferred_element_type=jnp.float32)
    h = jnp.maximum(h + b1_ref[...], 0.0)
    logits = jnp.dot(h, w2_ref[...], preferred_element_type=jnp.float32)
    logits = logits + b2_ref[...]
    lane = lax.broadcasted_iota(jnp.int32, logits.shape, 1)
    logits = jnp.where(lane < 2, logits, NEG_INF)
    mx = jnp.max(logits, axis=1, keepdims=True)
    e = jnp.exp(logits - mx)
    out_ref[...] = e / jnp.sum(e, axis=1, keepdims=True)


@functools.partial(jax.jit, static_argnames=("k",))
def _phrase_sim(outputs1, outputs2, mask1, mask2, W_s, b_s, gW1, gb1, gW2,
                gb2, k=128):
    len1, bsz, hdim = outputs1.shape
    len2 = outputs2.shape[0]
    nslices = W_s.shape[0]
    h1 = outputs1.transpose(1, 0, 2)
    h2 = outputs2.transpose(1, 0, 2)
    neg1 = jnp.where(mask1 == 0, 0.0, NEG_INF).astype(jnp.float32)
    neg1 = neg1.transpose(1, 0)[:, :, None]
    neg2 = jnp.where(mask2 == 0, 0.0, NEG_INF).astype(jnp.float32)
    neg2 = neg2.transpose(1, 0)[:, None, :]
    b_smem = b_s.reshape(nslices, 1, 1).astype(jnp.float32)

    grid = (nslices, bsz)
    q = pl.pallas_call(
        functools.partial(_topk_sim_kernel, k=k, len1=len1, len2=len2),
        grid=grid,
        in_specs=[
            pl.BlockSpec((1, len1, hdim), lambda s, b: (b, 0, 0)),
            pl.BlockSpec((1, len2, hdim), lambda s, b: (b, 0, 0)),
            pl.BlockSpec((1, hdim, hdim), lambda s, b: (s, 0, 0)),
            pl.BlockSpec((1, 1, 1), lambda s, b: (s, 0, 0),
                         memory_space=pltpu.SMEM),
            pl.BlockSpec((1, len1, 1), lambda s, b: (b, 0, 0)),
            pl.BlockSpec((1, 1, len2), lambda s, b: (b, 0, 0)),
        ],
        out_specs=pl.BlockSpec((1, 1, 1, k), lambda s, b: (b, s, 0, 0)),
        out_shape=jax.ShapeDtypeStruct((bsz, nslices, 1, k), jnp.float32),
        scratch_shapes=[
            pltpu.VMEM((len1, len2), jnp.float32),
            pltpu.VMEM((len1 // 128, 128), jnp.float32),
        ],
    )(h1, h2, W_s, b_smem, neg1, neg2)
    q = q.reshape(bsz, nslices * k)

    d = nslices * k
    w2p = jnp.zeros((d, 128), jnp.float32).at[:, :2].set(gW2)
    b2p = jnp.zeros((1, 128), jnp.float32).at[0, :2].set(gb2)
    probs = pl.pallas_call(
        _mlp_kernel,
        out_shape=jax.ShapeDtypeStruct((bsz, 128), jnp.float32),
    )(q, gW1, gb1.reshape(1, d), w2p, b2p)
    return probs[:, :2]


def kernel(outputs1, outputs2, mask1, mask2, W_s, b_s, gW1, gb1, gW2, gb2):
    return _phrase_sim(outputs1, outputs2, mask1, mask2, W_s, b_s, gW1, gb1,
                       gW2, gb2, k=128)


# speedup vs baseline: 12.7839x; 1.0122x over previous
import functools

import jax
import jax.numpy as jnp
from jax import lax
from jax.experimental import pallas as pl
from jax.experimental.pallas import tpu as pltpu

NEG_INF = float("-inf")


def _topk_sim_kernel(h1_ref, h2_ref, w_ref, b_ref, neg1_ref, neg2_ref,
                     out_ref, s_scr, p_scr, *, k, len1, len2, nslices):
    h1 = h1_ref[0]
    h2 = h2_ref[0]
    for sl in range(nslices):
        a = jnp.dot(h1, w_ref[sl], preferred_element_type=jnp.float32)
        s = lax.dot_general(a, h2, (((1,), (1,)), ((), ())),
                            preferred_element_type=jnp.float32)
        s = (jnp.maximum(s + b_ref[sl, 0, 0], 0.0)
             + neg1_ref[0] + neg2_ref[0, 0][None, :])
        s_scr[sl] = s
        p_scr[sl] = jnp.max(s.reshape(len1 // 128, 128, len2), axis=2)

    rows16 = len1 // 128
    iota_ph = lax.broadcasted_iota(jnp.int32, (rows16, 128), 0)
    iota_pl = lax.broadcasted_iota(jnp.int32, (rows16, 128), 1)
    iota_row = iota_ph * 128 + iota_pl
    iota_col = lax.broadcasted_iota(jnp.int32, (1, len2), 1)
    iota_p128 = lax.broadcasted_iota(jnp.int32, (1, 128), 1)
    iota_lane = lax.broadcasted_iota(jnp.int32, (1, k), 1)
    big = jnp.int32(len1 * len2)

    def body(i, _):
        for sl in range(nslices):
            p = p_scr[sl]
            m = jnp.max(p)
            r = jnp.min(jnp.where(p == m, iota_row, big))
            row = s_scr[sl, pl.ds(r, 1), :]
            j = jnp.min(jnp.where(row == m, iota_col, big))
            new_row = jnp.where(iota_col == j, NEG_INF, row)
            s_scr[sl, pl.ds(r, 1), :] = new_row
            new_max = jnp.max(new_row)
            rh = r // 128
            rl = r % 128
            prow = p_scr[sl, pl.ds(rh, 1), :]
            p_scr[sl, pl.ds(rh, 1), :] = jnp.where(iota_p128 == rl,
                                                   new_max, prow)
            out_ref[0, sl, 0:1, :] = jnp.where(iota_lane == i, m,
                                               out_ref[0, sl, 0:1, :])
        return 0

    lax.fori_loop(0, k, body, 0)

    for sl in range(nslices):
        q = out_ref[0, sl, 0:1, :]
        isneg = q == NEG_INF
        nf = k - jnp.sum(isneg.astype(jnp.int32))
        idx = lax.rem(lax.rem(nf - 1, k) + k, k)
        last_val = jnp.sum(jnp.where(iota_lane == idx, q, 0.0))
        out_ref[0, sl, 0:1, :] = jnp.where(isneg, last_val, q)


def _mlp_kernel(q_ref, w1_ref, b1_ref, w2_ref, b2_ref, out_ref):
    h = jnp.dot(q_ref[...], w1_ref[...], preferred_element_type=jnp.float32)
    h = jnp.maximum(h + b1_ref[...], 0.0)
    logits = jnp.dot(h, w2_ref[...], preferred_element_type=jnp.float32)
    logits = logits + b2_ref[...]
    lane = lax.broadcasted_iota(jnp.int32, logits.shape, 1)
    logits = jnp.where(lane < 2, logits, NEG_INF)
    mx = jnp.max(logits, axis=1, keepdims=True)
    e = jnp.exp(logits - mx)
    out_ref[...] = e / jnp.sum(e, axis=1, keepdims=True)


@functools.partial(jax.jit, static_argnames=("k",))
def _phrase_sim(outputs1, outputs2, mask1, mask2, W_s, b_s, gW1, gb1, gW2,
                gb2, k=128):
    len1, bsz, hdim = outputs1.shape
    len2 = outputs2.shape[0]
    nslices = W_s.shape[0]
    h1 = outputs1.transpose(1, 0, 2)
    h2 = outputs2.transpose(1, 0, 2)
    neg1 = jnp.where(mask1 == 0, 0.0, NEG_INF).astype(jnp.float32)
    neg1 = neg1.transpose(1, 0)[:, :, None]
    neg2 = jnp.where(mask2 == 0, 0.0, NEG_INF).astype(jnp.float32)
    neg2 = neg2.transpose(1, 0)[:, None, :]
    b_smem = b_s.reshape(nslices, 1, 1).astype(jnp.float32)

    grid = (bsz,)
    q = pl.pallas_call(
        functools.partial(_topk_sim_kernel, k=k, len1=len1, len2=len2,
                          nslices=nslices),
        grid=grid,
        in_specs=[
            pl.BlockSpec((1, len1, hdim), lambda b: (b, 0, 0)),
            pl.BlockSpec((1, len2, hdim), lambda b: (b, 0, 0)),
            pl.BlockSpec((nslices, hdim, hdim), lambda b: (0, 0, 0)),
            pl.BlockSpec((nslices, 1, 1), lambda b: (0, 0, 0),
                         memory_space=pltpu.SMEM),
            pl.BlockSpec((1, len1, 1), lambda b: (b, 0, 0)),
            pl.BlockSpec((1, 1, len2), lambda b: (b, 0, 0)),
        ],
        out_specs=pl.BlockSpec((1, nslices, 1, k), lambda b: (b, 0, 0, 0)),
        out_shape=jax.ShapeDtypeStruct((bsz, nslices, 1, k), jnp.float32),
        scratch_shapes=[
            pltpu.VMEM((nslices, len1, len2), jnp.float32),
            pltpu.VMEM((nslices, len1 // 128, 128), jnp.float32),
        ],
    )(h1, h2, W_s, b_smem, neg1, neg2)
    q = q.reshape(bsz, nslices * k)

    d = nslices * k
    w2p = jnp.zeros((d, 128), jnp.float32).at[:, :2].set(gW2)
    b2p = jnp.zeros((1, 128), jnp.float32).at[0, :2].set(gb2)
    probs = pl.pallas_call(
        _mlp_kernel,
        out_shape=jax.ShapeDtypeStruct((bsz, 128), jnp.float32),
    )(q, gW1, gb1.reshape(1, d), w2p, b2p)
    return probs[:, :2]


def kernel(outputs1, outputs2, mask1, mask2, W_s, b_s, gW1, gb1, gW2, gb2):
    return _phrase_sim(outputs1, outputs2, mask1, mask2, W_s, b_s, gW1, gb1,
                       gW2, gb2, k=128)


# speedup vs baseline: 13.9724x; 1.0930x over previous
import functools

import jax
import jax.numpy as jnp
from jax import lax
from jax.experimental import pallas as pl
from jax.experimental.pallas import tpu as pltpu

NEG_INF = float("-inf")


def _topk_sim_kernel(h1_ref, h2_ref, w_ref, b_ref, neg1_ref, neg2_ref,
                     out_ref, s_scr, p_scr, *, k, len1, len2, nslices):
    h1 = h1_ref[0]
    h2 = h2_ref[0]
    for sl in range(nslices):
        a = jnp.dot(h1, w_ref[sl], preferred_element_type=jnp.float32)
        s = lax.dot_general(a, h2, (((1,), (1,)), ((), ())),
                            preferred_element_type=jnp.float32)
        s = (jnp.maximum(s + b_ref[sl, 0, 0], 0.0)
             + neg1_ref[0] + neg2_ref[0, 0][None, :])
        s_scr[sl] = s.reshape(len1 // 8, 8, len2)
        p_scr[sl] = jnp.max(s.reshape(len1 // 128, 128, len2), axis=2)

    rows16 = len1 // 128
    iota_ph = lax.broadcasted_iota(jnp.int32, (rows16, 128), 0)
    iota_pl = lax.broadcasted_iota(jnp.int32, (rows16, 128), 1)
    iota_row = iota_ph * 128 + iota_pl
    iota_sub8 = lax.broadcasted_iota(jnp.int32, (8, len2), 0)
    iota_col = lax.broadcasted_iota(jnp.int32, (8, len2), 1)
    iota_flat = iota_sub8 * len2 + iota_col
    iota_lane = lax.broadcasted_iota(jnp.int32, (1, k), 1)
    big = jnp.int32(len1 * len2)

    def body(i, _):
        for sl in range(nslices):
            p = p_scr[sl]
            m = jnp.max(p)
            r = jnp.min(jnp.where(p == m, iota_row, big))
            rb = r // 8
            sub = r % 8
            blk = s_scr[sl, rb]
            hit = (blk == m) & (iota_sub8 == sub)
            jf = jnp.min(jnp.where(hit, iota_flat, big))
            new_blk = jnp.where(iota_flat == jf, NEG_INF, blk)
            s_scr[sl, rb] = new_blk
            new_max = jnp.max(jnp.where(iota_sub8 == sub, new_blk, NEG_INF))
            p_scr[sl] = jnp.where(iota_row == r, new_max, p)
            out_ref[0, sl, 0:1, :] = jnp.where(iota_lane == i, m,
                                               out_ref[0, sl, 0:1, :])
        return 0

    lax.fori_loop(0, k, body, 0)

    for sl in range(nslices):
        q = out_ref[0, sl, 0:1, :]
        isneg = q == NEG_INF
        nf = k - jnp.sum(isneg.astype(jnp.int32))
        idx = lax.rem(lax.rem(nf - 1, k) + k, k)
        last_val = jnp.sum(jnp.where(iota_lane == idx, q, 0.0))
        out_ref[0, sl, 0:1, :] = jnp.where(isneg, last_val, q)


def _mlp_kernel(q_ref, w1_ref, b1_ref, w2_ref, b2_ref, out_ref):
    h = jnp.dot(q_ref[...], w1_ref[...], preferred_element_type=jnp.float32)
    h = jnp.maximum(h + b1_ref[...], 0.0)
    logits = jnp.dot(h, w2_ref[...], preferred_element_type=jnp.float32)
    logits = logits + b2_ref[...]
    lane = lax.broadcasted_iota(jnp.int32, logits.shape, 1)
    logits = jnp.where(lane < 2, logits, NEG_INF)
    mx = jnp.max(logits, axis=1, keepdims=True)
    e = jnp.exp(logits - mx)
    out_ref[...] = e / jnp.sum(e, axis=1, keepdims=True)


@functools.partial(jax.jit, static_argnames=("k",))
def _phrase_sim(outputs1, outputs2, mask1, mask2, W_s, b_s, gW1, gb1, gW2,
                gb2, k=128):
    len1, bsz, hdim = outputs1.shape
    len2 = outputs2.shape[0]
    nslices = W_s.shape[0]
    h1 = outputs1.transpose(1, 0, 2)
    h2 = outputs2.transpose(1, 0, 2)
    neg1 = jnp.where(mask1 == 0, 0.0, NEG_INF).astype(jnp.float32)
    neg1 = neg1.transpose(1, 0)[:, :, None]
    neg2 = jnp.where(mask2 == 0, 0.0, NEG_INF).astype(jnp.float32)
    neg2 = neg2.transpose(1, 0)[:, None, :]
    b_smem = b_s.reshape(nslices, 1, 1).astype(jnp.float32)

    grid = (bsz,)
    q = pl.pallas_call(
        functools.partial(_topk_sim_kernel, k=k, len1=len1, len2=len2,
                          nslices=nslices),
        grid=grid,
        in_specs=[
            pl.BlockSpec((1, len1, hdim), lambda b: (b, 0, 0)),
            pl.BlockSpec((1, len2, hdim), lambda b: (b, 0, 0)),
            pl.BlockSpec((nslices, hdim, hdim), lambda b: (0, 0, 0)),
            pl.BlockSpec((nslices, 1, 1), lambda b: (0, 0, 0),
                         memory_space=pltpu.SMEM),
            pl.BlockSpec((1, len1, 1), lambda b: (b, 0, 0)),
            pl.BlockSpec((1, 1, len2), lambda b: (b, 0, 0)),
        ],
        out_specs=pl.BlockSpec((1, nslices, 1, k), lambda b: (b, 0, 0, 0)),
        out_shape=jax.ShapeDtypeStruct((bsz, nslices, 1, k), jnp.float32),
        scratch_shapes=[
            pltpu.VMEM((nslices, len1 // 8, 8, len2), jnp.float32),
            pltpu.VMEM((nslices, len1 // 128, 128), jnp.float32),
        ],
    )(h1, h2, W_s, b_smem, neg1, neg2)
    q = q.reshape(bsz, nslices * k)

    d = nslices * k
    w2p = jnp.zeros((d, 128), jnp.float32).at[:, :2].set(gW2)
    b2p = jnp.zeros((1, 128), jnp.float32).at[0, :2].set(gb2)
    probs = pl.pallas_call(
        _mlp_kernel,
        out_shape=jax.ShapeDtypeStruct((bsz, 128), jnp.float32),
    )(q, gW1, gb1.reshape(1, d), w2p, b2p)
    return probs[:, :2]


def kernel(outputs1, outputs2, mask1, mask2, W_s, b_s, gW1, gb1, gW2, gb2):
    return _phrase_sim(outputs1, outputs2, mask1, mask2, W_s, b_s, gW1, gb1,
                       gW2, gb2, k=128)
